# baseline (speedup 1.0000x reference)
import functools

import jax
import jax.numpy as jnp
from jax import lax
from jax.experimental import pallas as pl
from jax.experimental.pallas import tpu as pltpu
from jax.experimental.pallas import tpu_sc as plsc

B = 8192
D = 256
K = 8192

BM = 4096
BK = 512
NB = B // BM
NK = K // BK


def _argmin_body(z_ref, cb_ref, idx_ref, minv_ref):
    k = pl.program_id(1)

    @pl.when(k == 0)
    def _():
        minv_ref[...] = jnp.full((1, BM), jnp.inf, jnp.float32)
        idx_ref[...] = jnp.zeros((1, 1, BM), jnp.int32)

    zs = z_ref[...] * (-2.0)
    cb = cb_ref[...]
    c_sq = jnp.sum(cb * cb, axis=1, keepdims=True)
    scores = lax.dot_general(cb, zs, (((1,), (1,)), ((), ())),
                             preferred_element_type=jnp.float32) + c_sq
    val = scores
    idx = lax.broadcasted_iota(jnp.int32, scores.shape, 0) + k * BK
    h = BK // 2
    while h >= 8:
        keep = val[:h] <= val[h:]
        val = jnp.where(keep, val[:h], val[h:])
        idx = jnp.where(keep, idx[:h], idx[h:])
        h //= 2
    m = jnp.min(val, axis=0, keepdims=True)
    bidx = jnp.min(jnp.where(val == m, idx, K), axis=0, keepdims=True)
    better = m < minv_ref[...]
    minv_ref[...] = jnp.where(better, m, minv_ref[...])
    idx_ref[...] = jnp.where(better[None], bidx[None], idx_ref[...])


def _argmin_call(z, cb, off, nb):
    out = pl.pallas_call(
        _argmin_body,
        grid=(nb, NK),
        in_specs=[pl.BlockSpec((BM, D), lambda i, k, o=off: (i + o, 0)),
                  pl.BlockSpec((BK, D), lambda i, k: (k, 0))],
        out_specs=pl.BlockSpec((1, 1, BM), lambda i, k: (i, 0, 0)),
        out_shape=jax.ShapeDtypeStruct((nb, 1, BM), jnp.int32),
        scratch_shapes=[pltpu.VMEM((1, BM), jnp.float32)],
        compiler_params=pltpu.CompilerParams(
            dimension_semantics=("arbitrary", "arbitrary")),
    )(z, cb)
    return out.reshape(nb * BM)


SC_NC = 2
SC_NS = 16
NW = SC_NC * SC_NS
CHK = 128

@functools.cache
def _build_gather(rows):
    bpw = rows // NW
    nchk = bpw // CHK
    mesh = plsc.VectorSubcoreMesh(core_axis_name="c", subcore_axis_name="s")

    @functools.partial(
        pl.kernel, mesh=mesh,
        out_type=jax.ShapeDtypeStruct((rows, D), jnp.float32),
        scratch_types=[
            pltpu.VMEM((nchk, CHK), jnp.int32),
            pltpu.VMEM((bpw, D), jnp.float32),
            pltpu.SemaphoreType.DMA,
        ],
    )
    def _gather(idx_hbm, table_hbm, out_hbm, idx_v, rows_v, sem):
        wid = lax.axis_index("s") * SC_NC + lax.axis_index("c")
        base = wid * bpw
        pltpu.sync_copy(idx_hbm.at[pl.ds(wid * nchk, nchk)], idx_v)
        copies = [
            pltpu.async_copy(table_hbm.at[idx_v.at[j]],
                             rows_v.at[pl.ds(j * CHK, CHK)], sem)
            for j in range(nchk)
        ]
        for cp in copies:
            cp.wait()
        pltpu.sync_copy(rows_v, out_hbm.at[pl.ds(base, bpw)])

    return _gather


BC = 1024
NBC = B // BC


def _finish_core(z_ref, n_ref, v_ref, zq_ref, dist_ref):
    z = z_ref[...]
    d = n_ref[...] - z
    d2 = jnp.sum(d * d, axis=1, keepdims=True)
    dist = jnp.sqrt(d2)
    vd = v_ref[...] + d
    vn = jnp.sqrt(jnp.sum(vd * vd, axis=1, keepdims=True)) + 1e-8
    zq_ref[...] = z + vd * (dist / vn)
    dist_ref[...] = dist
    return jnp.sum(d2)


def _finish_body(z_ref, n_ref, v_ref, zq_ref, dist_ref, acc_ref):
    i = pl.program_id(0)
    s = _finish_core(z_ref, n_ref, v_ref, zq_ref, dist_ref)
    prev = jnp.where(i == 0, 0.0, acc_ref[0, 0])
    acc_ref[...] = jnp.full((1, 1), prev + s, jnp.float32)


def _finish_body_alias(z_ref, n_ref, v_ref, zqp_ref, dp_ref, acc0_ref,
                       zq_ref, dist_ref, acc_ref):
    del zqp_ref, dp_ref
    i = pl.program_id(0)
    s = _finish_core(z_ref, n_ref, v_ref, zq_ref, dist_ref)
    prev = jnp.where(i == 0, 0.0, acc_ref[0, 0])
    tot = prev + s
    tot = jnp.where(i == pl.num_programs(0) - 1,
                    (tot + acc0_ref[0, 0]) * jnp.float32(1.0 / (B * D)), tot)
    acc_ref[...] = jnp.full((1, 1), tot, jnp.float32)


def _finish_call(z, nearest, v, off, prev=None):
    half_rows = nearest.shape[0]
    zmap = lambda i, o=off: (i + o, 0)
    in_specs = [pl.BlockSpec((BC, D), zmap),
                pl.BlockSpec((BC, D), lambda i: (i, 0)),
                pl.BlockSpec((BC, D), zmap)]
    args = [z, nearest, v]
    aliases = {}
    body = _finish_body
    if prev is not None:
        in_specs += [pl.BlockSpec(memory_space=pl.ANY),
                     pl.BlockSpec(memory_space=pl.ANY),
                     pl.BlockSpec((1, 1), lambda i: (0, 0))]
        args += [prev[0], prev[1], prev[2]]
        aliases = {3: 0, 4: 1}
        body = _finish_body_alias
    return pl.pallas_call(
        body,
        grid=(half_rows // BC,),
        in_specs=in_specs,
        out_specs=[pl.BlockSpec((BC, D), zmap),
                   pl.BlockSpec((BC, 1), zmap),
                   pl.BlockSpec((1, 1), lambda i: (0, 0))],
        out_shape=[jax.ShapeDtypeStruct((B, D), jnp.float32),
                   jax.ShapeDtypeStruct((B, 1), jnp.float32),
                   jax.ShapeDtypeStruct((1, 1), jnp.float32)],
        input_output_aliases=aliases,
        compiler_params=pltpu.CompilerParams(
            dimension_semantics=("arbitrary",)),
    )(*args)


HALVES = 2
HR = B // HALVES


def kernel(z, codebook, v):
    idx_h = [_argmin_call(z, codebook, h * (HR // BM), HR // BM)
             for h in range(HALVES)]
    gather = _build_gather(HR)
    near_h = [gather(idx_h[h].reshape(HR // CHK, CHK), codebook)
              for h in range(HALVES)]
    fin0 = _finish_call(z, near_h[0], v, 0)
    z_q, dist2d, acc1 = _finish_call(z, near_h[1], v, HR // BC, fin0)
    indices = jnp.concatenate(idx_h, axis=0)
    dist = dist2d.reshape(B)
    commit_loss = acc1[0, 0]
    return (z_q, indices, dist, commit_loss)


# speedup vs baseline: 1.0314x; 1.0314x over previous
import functools

import jax
import jax.numpy as jnp
from jax import lax
from jax.experimental import pallas as pl
from jax.experimental.pallas import tpu as pltpu
from jax.experimental.pallas import tpu_sc as plsc

B = 8192
D = 256
K = 8192

BM = 2048
BK = 1024
NB = B // BM
NK = K // BK


def _argmin_body(z_ref, cb_ref, idx_ref, minv_ref):
    k = pl.program_id(1)

    @pl.when(k == 0)
    def _():
        minv_ref[...] = jnp.full((1, BM), jnp.inf, jnp.float32)
        idx_ref[...] = jnp.zeros((1, 1, BM), jnp.int32)

    zs = z_ref[...] * (-2.0)
    cb = cb_ref[...]
    c_sq = jnp.sum(cb * cb, axis=1, keepdims=True)
    scores = lax.dot_general(cb, zs, (((1,), (1,)), ((), ())),
                             preferred_element_type=jnp.float32) + c_sq
    val = scores
    idx = lax.broadcasted_iota(jnp.int32, scores.shape, 0) + k * BK
    h = BK // 2
    while h >= 8:
        keep = val[:h] <= val[h:]
        val = jnp.where(keep, val[:h], val[h:])
        idx = jnp.where(keep, idx[:h], idx[h:])
        h //= 2
    m = jnp.min(val, axis=0, keepdims=True)
    bidx = jnp.min(jnp.where(val == m, idx, K), axis=0, keepdims=True)
    better = m < minv_ref[...]
    minv_ref[...] = jnp.where(better, m, minv_ref[...])
    idx_ref[...] = jnp.where(better[None], bidx[None], idx_ref[...])


def _argmin_call(z, cb, off, nb):
    out = pl.pallas_call(
        _argmin_body,
        grid=(nb, NK),
        in_specs=[pl.BlockSpec((BM, D), lambda i, k, o=off: (i + o, 0)),
                  pl.BlockSpec((BK, D), lambda i, k: (k, 0))],
        out_specs=pl.BlockSpec((1, 1, BM), lambda i, k: (i, 0, 0)),
        out_shape=jax.ShapeDtypeStruct((nb, 1, BM), jnp.int32),
        scratch_shapes=[pltpu.VMEM((1, BM), jnp.float32)],
        compiler_params=pltpu.CompilerParams(
            dimension_semantics=("arbitrary", "arbitrary")),
    )(z, cb)
    return out.reshape(nb * BM)


SC_NC = 2
SC_NS = 16
NW = SC_NC * SC_NS
CHK = 128

@functools.cache
def _build_gather(rows):
    bpw = rows // NW
    nchk = bpw // CHK
    mesh = plsc.VectorSubcoreMesh(core_axis_name="c", subcore_axis_name="s")

    @functools.partial(
        pl.kernel, mesh=mesh,
        out_type=jax.ShapeDtypeStruct((rows, D), jnp.float32),
        scratch_types=[
            pltpu.VMEM((nchk, CHK), jnp.int32),
            pltpu.VMEM((bpw, D), jnp.float32),
            pltpu.SemaphoreType.DMA,
        ],
    )
    def _gather(idx_hbm, table_hbm, out_hbm, idx_v, rows_v, sem):
        wid = lax.axis_index("s") * SC_NC + lax.axis_index("c")
        base = wid * bpw
        pltpu.sync_copy(idx_hbm.at[pl.ds(wid * nchk, nchk)], idx_v)
        copies = [
            pltpu.async_copy(table_hbm.at[idx_v.at[j]],
                             rows_v.at[pl.ds(j * CHK, CHK)], sem)
            for j in range(nchk)
        ]
        for cp in copies:
            cp.wait()
        pltpu.sync_copy(rows_v, out_hbm.at[pl.ds(base, bpw)])

    return _gather


BC = 1024
NBC = B // BC


def _finish_core(z_ref, n_ref, v_ref, zq_ref, dist_ref):
    z = z_ref[...]
    d = n_ref[...] - z
    d2 = jnp.sum(d * d, axis=1, keepdims=True)
    dist = jnp.sqrt(d2)
    vd = v_ref[...] + d
    vn = jnp.sqrt(jnp.sum(vd * vd, axis=1, keepdims=True)) + 1e-8
    zq_ref[...] = z + vd * (dist / vn)
    dist_ref[...] = dist
    return jnp.sum(d2)


def _finish_body(z_ref, n_ref, v_ref, zq_ref, dist_ref, acc_ref):
    i = pl.program_id(0)
    s = _finish_core(z_ref, n_ref, v_ref, zq_ref, dist_ref)
    prev = jnp.where(i == 0, 0.0, acc_ref[0, 0])
    acc_ref[...] = jnp.full((1, 1), prev + s, jnp.float32)


def _finish_body_alias(z_ref, n_ref, v_ref, zqp_ref, dp_ref, acc0_ref,
                       zq_ref, dist_ref, acc_ref):
    del zqp_ref, dp_ref
    i = pl.program_id(0)
    s = _finish_core(z_ref, n_ref, v_ref, zq_ref, dist_ref)
    prev = jnp.where(i == 0, 0.0, acc_ref[0, 0])
    tot = prev + s
    tot = jnp.where(i == pl.num_programs(0) - 1,
                    (tot + acc0_ref[0, 0]) * jnp.float32(1.0 / (B * D)), tot)
    acc_ref[...] = jnp.full((1, 1), tot, jnp.float32)


def _finish_call(z, nearest, v, off, prev=None):
    half_rows = nearest.shape[0]
    zmap = lambda i, o=off: (i + o, 0)
    in_specs = [pl.BlockSpec((BC, D), zmap),
                pl.BlockSpec((BC, D), lambda i: (i, 0)),
                pl.BlockSpec((BC, D), zmap)]
    args = [z, nearest, v]
    aliases = {}
    body = _finish_body
    if prev is not None:
        in_specs += [pl.BlockSpec(memory_space=pl.ANY),
                     pl.BlockSpec(memory_space=pl.ANY),
                     pl.BlockSpec((1, 1), lambda i: (0, 0))]
        args += [prev[0], prev[1], prev[2]]
        aliases = {3: 0, 4: 1}
        body = _finish_body_alias
    return pl.pallas_call(
        body,
        grid=(half_rows // BC,),
        in_specs=in_specs,
        out_specs=[pl.BlockSpec((BC, D), zmap),
                   pl.BlockSpec((BC, 1), zmap),
                   pl.BlockSpec((1, 1), lambda i: (0, 0))],
        out_shape=[jax.ShapeDtypeStruct((B, D), jnp.float32),
                   jax.ShapeDtypeStruct((B, 1), jnp.float32),
                   jax.ShapeDtypeStruct((1, 1), jnp.float32)],
        input_output_aliases=aliases,
        compiler_params=pltpu.CompilerParams(
            dimension_semantics=("arbitrary",)),
    )(*args)


HALVES = 2
HR = B // HALVES


def kernel(z, codebook, v):
    idx_h = [_argmin_call(z, codebook, h * (HR // BM), HR // BM)
             for h in range(HALVES)]
    gather = _build_gather(HR)
    near_h = [gather(idx_h[h].reshape(HR // CHK, CHK), codebook)
              for h in range(HALVES)]
    fin0 = _finish_call(z, near_h[0], v, 0)
    z_q, dist2d, acc1 = _finish_call(z, near_h[1], v, HR // BC, fin0)
    indices = jnp.concatenate(idx_h, axis=0)
    dist = dist2d.reshape(B)
    commit_loss = acc1[0, 0]
    return (z_q, indices, dist, commit_loss)


# speedup vs baseline: 1.0776x; 1.0447x over previous
import functools

import jax
import jax.numpy as jnp
from jax import lax
from jax.experimental import pallas as pl
from jax.experimental.pallas import tpu as pltpu
from jax.experimental.pallas import tpu_sc as plsc

B = 8192
D = 256
K = 8192

BM = 2048
BK = 2048
NB = B // BM
NK = K // BK


def _argmin_body(z_ref, cb_ref, idx_ref, minv_ref):
    k = pl.program_id(1)

    @pl.when(k == 0)
    def _():
        minv_ref[...] = jnp.full((1, BM), jnp.inf, jnp.float32)
        idx_ref[...] = jnp.zeros((1, 1, BM), jnp.int32)

    zs = z_ref[...] * (-2.0)
    cb = cb_ref[...]
    c_sq = jnp.sum(cb * cb, axis=1, keepdims=True)
    scores = lax.dot_general(cb, zs, (((1,), (1,)), ((), ())),
                             preferred_element_type=jnp.float32) + c_sq
    val = scores
    idx = lax.broadcasted_iota(jnp.int32, scores.shape, 0) + k * BK
    h = BK // 2
    while h >= 8:
        keep = val[:h] <= val[h:]
        val = jnp.where(keep, val[:h], val[h:])
        idx = jnp.where(keep, idx[:h], idx[h:])
        h //= 2
    m = jnp.min(val, axis=0, keepdims=True)
    bidx = jnp.min(jnp.where(val == m, idx, K), axis=0, keepdims=True)
    better = m < minv_ref[...]
    minv_ref[...] = jnp.where(better, m, minv_ref[...])
    idx_ref[...] = jnp.where(better[None], bidx[None], idx_ref[...])


def _argmin_call(z, cb, off, nb):
    out = pl.pallas_call(
        _argmin_body,
        grid=(nb, NK),
        in_specs=[pl.BlockSpec((BM, D), lambda i, k, o=off: (i + o, 0)),
                  pl.BlockSpec((BK, D), lambda i, k: (k, 0))],
        out_specs=pl.BlockSpec((1, 1, BM), lambda i, k: (i, 0, 0)),
        out_shape=jax.ShapeDtypeStruct((nb, 1, BM), jnp.int32),
        scratch_shapes=[pltpu.VMEM((1, BM), jnp.float32)],
        compiler_params=pltpu.CompilerParams(
            dimension_semantics=("arbitrary", "arbitrary")),
    )(z, cb)
    return out.reshape(nb * BM)


SC_NC = 2
SC_NS = 16
NW = SC_NC * SC_NS
CHK = 128

@functools.cache
def _build_gather(rows):
    bpw = rows // NW
    nchk = bpw // CHK
    mesh = plsc.VectorSubcoreMesh(core_axis_name="c", subcore_axis_name="s")

    @functools.partial(
        pl.kernel, mesh=mesh,
        out_type=jax.ShapeDtypeStruct((rows, D), jnp.float32),
        scratch_types=[
            pltpu.VMEM((nchk, CHK), jnp.int32),
            pltpu.VMEM((bpw, D), jnp.float32),
            pltpu.SemaphoreType.DMA,
        ],
    )
    def _gather(idx_hbm, table_hbm, out_hbm, idx_v, rows_v, sem):
        wid = lax.axis_index("s") * SC_NC + lax.axis_index("c")
        base = wid * bpw
        pltpu.sync_copy(idx_hbm.at[pl.ds(wid * nchk, nchk)], idx_v)
        copies = [
            pltpu.async_copy(table_hbm.at[idx_v.at[j]],
                             rows_v.at[pl.ds(j * CHK, CHK)], sem)
            for j in range(nchk)
        ]
        for cp in copies:
            cp.wait()
        pltpu.sync_copy(rows_v, out_hbm.at[pl.ds(base, bpw)])

    return _gather


BC = 1024
NBC = B // BC


def _finish_core(z_ref, n_ref, v_ref, zq_ref, dist_ref):
    z = z_ref[...]
    d = n_ref[...] - z
    d2 = jnp.sum(d * d, axis=1, keepdims=True)
    dist = jnp.sqrt(d2)
    vd = v_ref[...] + d
    vn = jnp.sqrt(jnp.sum(vd * vd, axis=1, keepdims=True)) + 1e-8
    zq_ref[...] = z + vd * (dist / vn)
    dist_ref[...] = dist
    return jnp.sum(d2)


def _finish_body(z_ref, n_ref, v_ref, zq_ref, dist_ref, acc_ref):
    i = pl.program_id(0)
    s = _finish_core(z_ref, n_ref, v_ref, zq_ref, dist_ref)
    prev = jnp.where(i == 0, 0.0, acc_ref[0, 0])
    acc_ref[...] = jnp.full((1, 1), prev + s, jnp.float32)


def _finish_body_alias(z_ref, n_ref, v_ref, zqp_ref, dp_ref, acc0_ref,
                       zq_ref, dist_ref, acc_ref):
    del zqp_ref, dp_ref
    i = pl.program_id(0)
    s = _finish_core(z_ref, n_ref, v_ref, zq_ref, dist_ref)
    prev = jnp.where(i == 0, 0.0, acc_ref[0, 0])
    tot = prev + s
    tot = jnp.where(i == pl.num_programs(0) - 1,
                    (tot + acc0_ref[0, 0]) * jnp.float32(1.0 / (B * D)), tot)
    acc_ref[...] = jnp.full((1, 1), tot, jnp.float32)


def _finish_call(z, nearest, v, off, prev=None):
    half_rows = nearest.shape[0]
    zmap = lambda i, o=off: (i + o, 0)
    in_specs = [pl.BlockSpec((BC, D), zmap),
                pl.BlockSpec((BC, D), lambda i: (i, 0)),
                pl.BlockSpec((BC, D), zmap)]
    args = [z, nearest, v]
    aliases = {}
    body = _finish_body
    if prev is not None:
        in_specs += [pl.BlockSpec(memory_space=pl.ANY),
                     pl.BlockSpec(memory_space=pl.ANY),
                     pl.BlockSpec((1, 1), lambda i: (0, 0))]
        args += [prev[0], prev[1], prev[2]]
        aliases = {3: 0, 4: 1}
        body = _finish_body_alias
    return pl.pallas_call(
        body,
        grid=(half_rows // BC,),
        in_specs=in_specs,
        out_specs=[pl.BlockSpec((BC, D), zmap),
                   pl.BlockSpec((BC, 1), zmap),
                   pl.BlockSpec((1, 1), lambda i: (0, 0))],
        out_shape=[jax.ShapeDtypeStruct((B, D), jnp.float32),
                   jax.ShapeDtypeStruct((B, 1), jnp.float32),
                   jax.ShapeDtypeStruct((1, 1), jnp.float32)],
        input_output_aliases=aliases,
        compiler_params=pltpu.CompilerParams(
            dimension_semantics=("arbitrary",)),
    )(*args)


HALVES = 2
HR = B // HALVES


def kernel(z, codebook, v):
    idx_h = [_argmin_call(z, codebook, h * (HR // BM), HR // BM)
             for h in range(HALVES)]
    gather = _build_gather(HR)
    near_h = [gather(idx_h[h].reshape(HR // CHK, CHK), codebook)
              for h in range(HALVES)]
    fin0 = _finish_call(z, near_h[0], v, 0)
    z_q, dist2d, acc1 = _finish_call(z, near_h[1], v, HR // BC, fin0)
    indices = jnp.concatenate(idx_h, axis=0)
    dist = dist2d.reshape(B)
    commit_loss = acc1[0, 0]
    return (z_q, indices, dist, commit_loss)


# speedup vs baseline: 1.0873x; 1.0091x over previous
import functools

import jax
import jax.numpy as jnp
from jax import lax
from jax.experimental import pallas as pl
from jax.experimental.pallas import tpu as pltpu
from jax.experimental.pallas import tpu_sc as plsc

B = 8192
D = 256
K = 8192

BM = 2048
BK = 2048
NB = B // BM
NK = K // BK


def _argmin_body(z_ref, cb_ref, idx_ref, minv_ref):
    k = pl.program_id(1)

    @pl.when(k == 0)
    def _():
        minv_ref[...] = jnp.full((1, BM), jnp.inf, jnp.float32)
        idx_ref[...] = jnp.zeros((1, 1, BM), jnp.int32)

    zs = z_ref[...] * (-2.0)
    cb = cb_ref[...]
    c_sq = jnp.sum(cb * cb, axis=1, keepdims=True)
    scores = lax.dot_general(cb, zs, (((1,), (1,)), ((), ())),
                             preferred_element_type=jnp.float32) + c_sq
    val = scores
    idx = lax.broadcasted_iota(jnp.int32, scores.shape, 0)
    h = BK // 2
    while h >= 8:
        keep = val[:h] <= val[h:]
        val = jnp.where(keep, val[:h], val[h:])
        idx = jnp.where(keep, idx[:h], idx[h:])
        h //= 2
    m = jnp.min(val, axis=0, keepdims=True)
    bidx = jnp.min(jnp.where(val == m, idx, BK), axis=0,
                   keepdims=True) + k * BK
    better = m < minv_ref[...]
    minv_ref[...] = jnp.where(better, m, minv_ref[...])
    idx_ref[...] = jnp.where(better[None], bidx[None], idx_ref[...])


def _argmin_call(z, cb, off, nb):
    out = pl.pallas_call(
        _argmin_body,
        grid=(nb, NK),
        in_specs=[pl.BlockSpec((BM, D), lambda i, k, o=off: (i + o, 0)),
                  pl.BlockSpec((BK, D), lambda i, k: (k, 0))],
        out_specs=pl.BlockSpec((1, 1, BM), lambda i, k: (i, 0, 0)),
        out_shape=jax.ShapeDtypeStruct((nb, 1, BM), jnp.int32),
        scratch_shapes=[pltpu.VMEM((1, BM), jnp.float32)],
        compiler_params=pltpu.CompilerParams(
            dimension_semantics=("arbitrary", "arbitrary")),
    )(z, cb)
    return out.reshape(nb * BM)


SC_NC = 2
SC_NS = 16
NW = SC_NC * SC_NS
CHK = 128

@functools.cache
def _build_gather(rows):
    bpw = rows // NW
    nchk = bpw // CHK
    mesh = plsc.VectorSubcoreMesh(core_axis_name="c", subcore_axis_name="s")

    @functools.partial(
        pl.kernel, mesh=mesh,
        out_type=jax.ShapeDtypeStruct((rows, D), jnp.float32),
        scratch_types=[
            pltpu.VMEM((nchk, CHK), jnp.int32),
            pltpu.VMEM((bpw, D), jnp.float32),
            pltpu.SemaphoreType.DMA,
        ],
    )
    def _gather(idx_hbm, table_hbm, out_hbm, idx_v, rows_v, sem):
        wid = lax.axis_index("s") * SC_NC + lax.axis_index("c")
        base = wid * bpw
        pltpu.sync_copy(idx_hbm.at[pl.ds(wid * nchk, nchk)], idx_v)
        copies = [
            pltpu.async_copy(table_hbm.at[idx_v.at[j]],
                             rows_v.at[pl.ds(j * CHK, CHK)], sem)
            for j in range(nchk)
        ]
        for cp in copies:
            cp.wait()
        pltpu.sync_copy(rows_v, out_hbm.at[pl.ds(base, bpw)])

    return _gather


BC = 2048
NBC = B // BC


def _finish_core(z_ref, n_ref, v_ref, zq_ref, dist_ref):
    z = z_ref[...]
    d = n_ref[...] - z
    d2 = jnp.sum(d * d, axis=1, keepdims=True)
    dist = jnp.sqrt(d2)
    vd = v_ref[...] + d
    vn = jnp.sqrt(jnp.sum(vd * vd, axis=1, keepdims=True)) + 1e-8
    zq_ref[...] = z + vd * (dist / vn)
    dist_ref[...] = dist
    return jnp.sum(d2)


def _finish_body(z_ref, n_ref, v_ref, zq_ref, dist_ref, acc_ref):
    i = pl.program_id(0)
    s = _finish_core(z_ref, n_ref, v_ref, zq_ref, dist_ref)
    prev = jnp.where(i == 0, 0.0, acc_ref[0, 0])
    acc_ref[...] = jnp.full((1, 1), prev + s, jnp.float32)


def _finish_body_alias(z_ref, n_ref, v_ref, zqp_ref, dp_ref, acc0_ref,
                       zq_ref, dist_ref, acc_ref):
    del zqp_ref, dp_ref
    i = pl.program_id(0)
    s = _finish_core(z_ref, n_ref, v_ref, zq_ref, dist_ref)
    prev = jnp.where(i == 0, 0.0, acc_ref[0, 0])
    tot = prev + s
    tot = jnp.where(i == pl.num_programs(0) - 1,
                    (tot + acc0_ref[0, 0]) * jnp.float32(1.0 / (B * D)), tot)
    acc_ref[...] = jnp.full((1, 1), tot, jnp.float32)


def _finish_call(z, nearest, v, off, prev=None):
    half_rows = nearest.shape[0]
    zmap = lambda i, o=off: (i + o, 0)
    in_specs = [pl.BlockSpec((BC, D), zmap),
                pl.BlockSpec((BC, D), lambda i: (i, 0)),
                pl.BlockSpec((BC, D), zmap)]
    args = [z, nearest, v]
    aliases = {}
    body = _finish_body
    if prev is not None:
        in_specs += [pl.BlockSpec(memory_space=pl.ANY),
                     pl.BlockSpec(memory_space=pl.ANY),
                     pl.BlockSpec((1, 1), lambda i: (0, 0))]
        args += [prev[0], prev[1], prev[2]]
        aliases = {3: 0, 4: 1}
        body = _finish_body_alias
    return pl.pallas_call(
        body,
        grid=(half_rows // BC,),
        in_specs=in_specs,
        out_specs=[pl.BlockSpec((BC, D), zmap),
                   pl.BlockSpec((BC, 1), zmap),
                   pl.BlockSpec((1, 1), lambda i: (0, 0))],
        out_shape=[jax.ShapeDtypeStruct((B, D), jnp.float32),
                   jax.ShapeDtypeStruct((B, 1), jnp.float32),
                   jax.ShapeDtypeStruct((1, 1), jnp.float32)],
        input_output_aliases=aliases,
        compiler_params=pltpu.CompilerParams(
            dimension_semantics=("arbitrary",)),
    )(*args)


HALVES = 2
HR = B // HALVES


def kernel(z, codebook, v):
    idx_h = [_argmin_call(z, codebook, h * (HR // BM), HR // BM)
             for h in range(HALVES)]
    gather = _build_gather(HR)
    near_h = [gather(idx_h[h].reshape(HR // CHK, CHK), codebook)
              for h in range(HALVES)]
    fin0 = _finish_call(z, near_h[0], v, 0)
    z_q, dist2d, acc1 = _finish_call(z, near_h[1], v, HR // BC, fin0)
    indices = jnp.concatenate(idx_h, axis=0)
    dist = dist2d.reshape(B)
    commit_loss = acc1[0, 0]
    return (z_q, indices, dist, commit_loss)


# speedup vs baseline: 1.1275x; 1.0369x over previous
import functools

import jax
import jax.numpy as jnp
from jax import lax
from jax.experimental import pallas as pl
from jax.experimental.pallas import tpu as pltpu
from jax.experimental.pallas import tpu_sc as plsc

B = 8192
D = 256
K = 8192

BM = 2048
BK = 2048
NB = B // BM
NK = K // BK


def _argmin_body(z_ref, cb_ref, idx_ref, minv_ref):
    k = pl.program_id(1)

    @pl.when(k == 0)
    def _():
        minv_ref[...] = jnp.full((1, BM), jnp.inf, jnp.float32)
        idx_ref[...] = jnp.zeros((1, 1, BM), jnp.int32)

    zs = z_ref[...] * (-2.0)
    cb = cb_ref[...]
    c_sq = jnp.sum(cb * cb, axis=1, keepdims=True)
    scores = lax.dot_general(cb, zs, (((1,), (1,)), ((), ())),
                             preferred_element_type=jnp.float32) + c_sq
    val = scores
    idx = lax.broadcasted_iota(jnp.int32, scores.shape, 0)
    h = BK // 2
    while h >= 8:
        keep = val[:h] <= val[h:]
        val = jnp.where(keep, val[:h], val[h:])
        idx = jnp.where(keep, idx[:h], idx[h:])
        h //= 2
    m = jnp.min(val, axis=0, keepdims=True)
    bidx = jnp.min(jnp.where(val == m, idx, BK), axis=0,
                   keepdims=True) + k * BK
    better = m < minv_ref[...]
    minv_ref[...] = jnp.where(better, m, minv_ref[...])
    idx_ref[...] = jnp.where(better[None], bidx[None], idx_ref[...])


def _argmin_call(z, cb, off, nb):
    out = pl.pallas_call(
        _argmin_body,
        grid=(nb, NK),
        in_specs=[pl.BlockSpec((BM, D), lambda i, k, o=off: (i + o, 0)),
                  pl.BlockSpec((BK, D), lambda i, k: (k, 0))],
        out_specs=pl.BlockSpec((1, 1, BM), lambda i, k: (i, 0, 0)),
        out_shape=jax.ShapeDtypeStruct((nb, 1, BM), jnp.int32),
        scratch_shapes=[pltpu.VMEM((1, BM), jnp.float32)],
        compiler_params=pltpu.CompilerParams(
            dimension_semantics=("arbitrary", "arbitrary")),
    )(z, cb)
    return out.reshape(nb * BM)


SC_NC = 2
SC_NS = 16
NW = SC_NC * SC_NS
CHK = 128

@functools.cache
def _build_gather(rows):
    bpw = rows // NW
    nchk = bpw // CHK
    mesh = plsc.VectorSubcoreMesh(core_axis_name="c", subcore_axis_name="s")

    @functools.partial(
        pl.kernel, mesh=mesh,
        out_type=jax.ShapeDtypeStruct((rows, D), jnp.float32),
        scratch_types=[
            pltpu.VMEM((nchk, CHK), jnp.int32),
            pltpu.VMEM((bpw, D), jnp.float32),
            pltpu.SemaphoreType.DMA,
        ],
    )
    def _gather(idx_hbm, table_hbm, out_hbm, idx_v, rows_v, sem):
        wid = lax.axis_index("s") * SC_NC + lax.axis_index("c")
        base = wid * bpw
        pltpu.sync_copy(idx_hbm.at[pl.ds(wid * nchk, nchk)], idx_v)
        copies = [
            pltpu.async_copy(table_hbm.at[idx_v.at[j]],
                             rows_v.at[pl.ds(j * CHK, CHK)], sem)
            for j in range(nchk)
        ]
        for cp in copies:
            cp.wait()
        pltpu.sync_copy(rows_v, out_hbm.at[pl.ds(base, bpw)])

    return _gather


BC = 2048
NBC = B // BC


def _finish_core(z_ref, n_ref, v_ref, zq_ref, dist_ref):
    z = z_ref[...]
    d = n_ref[...] - z
    d2 = jnp.sum(d * d, axis=1, keepdims=True)
    dist = jnp.sqrt(d2)
    vd = v_ref[...] + d
    vn = jnp.sqrt(jnp.sum(vd * vd, axis=1, keepdims=True)) + 1e-8
    zq_ref[...] = z + vd * (dist / vn)
    dist_ref[...] = dist.reshape(BC)
    return jnp.sum(d2)


def _finish_body(z_ref, n_ref, v_ref, zq_ref, dist_ref, acc_ref):
    i = pl.program_id(0)
    s = _finish_core(z_ref, n_ref, v_ref, zq_ref, dist_ref)
    prev = jnp.where(i == 0, 0.0, acc_ref[0, 0])
    acc_ref[...] = jnp.full((1, 1), prev + s, jnp.float32)


def _finish_body_alias(z_ref, n_ref, v_ref, zqp_ref, dp_ref, acc0_ref,
                       zq_ref, dist_ref, acc_ref):
    del zqp_ref, dp_ref
    i = pl.program_id(0)
    s = _finish_core(z_ref, n_ref, v_ref, zq_ref, dist_ref)
    prev = jnp.where(i == 0, 0.0, acc_ref[0, 0])
    tot = prev + s
    tot = jnp.where(i == pl.num_programs(0) - 1,
                    (tot + acc0_ref[0, 0]) * jnp.float32(1.0 / (B * D)), tot)
    acc_ref[...] = jnp.full((1, 1), tot, jnp.float32)


def _finish_call(z, nearest, v, off, prev=None):
    half_rows = nearest.shape[0]
    zmap = lambda i, o=off: (i + o, 0)
    in_specs = [pl.BlockSpec((BC, D), zmap),
                pl.BlockSpec((BC, D), lambda i: (i, 0)),
                pl.BlockSpec((BC, D), zmap)]
    args = [z, nearest, v]
    aliases = {}
    body = _finish_body
    if prev is not None:
        in_specs += [pl.BlockSpec(memory_space=pl.ANY),
                     pl.BlockSpec(memory_space=pl.ANY),
                     pl.BlockSpec((1, 1), lambda i: (0, 0))]
        args += [prev[0], prev[1], prev[2]]
        aliases = {3: 0, 4: 1}
        body = _finish_body_alias
    return pl.pallas_call(
        body,
        grid=(half_rows // BC,),
        in_specs=in_specs,
        out_specs=[pl.BlockSpec((BC, D), zmap),
                   pl.BlockSpec((BC,), lambda i, o=off: (i + o,)),
                   pl.BlockSpec((1, 1), lambda i: (0, 0))],
        out_shape=[jax.ShapeDtypeStruct((B, D), jnp.float32),
                   jax.ShapeDtypeStruct((B,), jnp.float32),
                   jax.ShapeDtypeStruct((1, 1), jnp.float32)],
        input_output_aliases=aliases,
        compiler_params=pltpu.CompilerParams(
            dimension_semantics=("arbitrary",)),
    )(*args)


HALVES = 2
HR = B // HALVES


def kernel(z, codebook, v):
    idx_h = [_argmin_call(z, codebook, h * (HR // BM), HR // BM)
             for h in range(HALVES)]
    gather = _build_gather(HR)
    near_h = [gather(idx_h[h].reshape(HR // CHK, CHK), codebook)
              for h in range(HALVES)]
    fin0 = _finish_call(z, near_h[0], v, 0)
    z_q, dist, acc1 = _finish_call(z, near_h[1], v, HR // BC, fin0)
    indices = jnp.concatenate(idx_h, axis=0)
    commit_loss = acc1[0, 0]
    return (z_q, indices, dist, commit_loss)


# speedup vs baseline: 1.1370x; 1.0084x over previous
import functools

import jax
import jax.numpy as jnp
from jax import lax
from jax.experimental import pallas as pl
from jax.experimental.pallas import tpu as pltpu
from jax.experimental.pallas import tpu_sc as plsc

B = 8192
D = 256
K = 8192

BM = 2048
BK = 2048
NB = B // BM
NK = K // BK


def _argmin_body(z_ref, cb_ref, idx_ref, minv_ref):
    k = pl.program_id(1)

    @pl.when(k == 0)
    def _():
        minv_ref[...] = jnp.full((1, BM), -jnp.inf, jnp.float32)
        idx_ref[...] = jnp.zeros((1, 1, BM), jnp.int32)

    cb = cb_ref[...]
    c_half = jnp.sum(cb * cb, axis=1, keepdims=True) * 0.5
    scores = lax.dot_general(cb, z_ref[...], (((1,), (1,)), ((), ())),
                             preferred_element_type=jnp.float32) - c_half
    val = scores
    idx = lax.broadcasted_iota(jnp.int32, scores.shape, 0)
    h = BK // 2
    while h >= 8:
        keep = val[:h] >= val[h:]
        val = jnp.where(keep, val[:h], val[h:])
        idx = jnp.where(keep, idx[:h], idx[h:])
        h //= 2
    m = jnp.max(val, axis=0, keepdims=True)
    bidx = jnp.min(jnp.where(val == m, idx, BK), axis=0,
                   keepdims=True) + k * BK
    better = m > minv_ref[...]
    minv_ref[...] = jnp.where(better, m, minv_ref[...])
    idx_ref[...] = jnp.where(better[None], bidx[None], idx_ref[...])


def _argmin_call(z, cb, off, nb):
    out = pl.pallas_call(
        _argmin_body,
        grid=(nb, NK),
        in_specs=[pl.BlockSpec((BM, D), lambda i, k, o=off: (i + o, 0)),
                  pl.BlockSpec((BK, D), lambda i, k: (k, 0))],
        out_specs=pl.BlockSpec((1, 1, BM), lambda i, k: (i, 0, 0)),
        out_shape=jax.ShapeDtypeStruct((nb, 1, BM), jnp.int32),
        scratch_shapes=[pltpu.VMEM((1, BM), jnp.float32)],
        compiler_params=pltpu.CompilerParams(
            dimension_semantics=("arbitrary", "arbitrary")),
    )(z, cb)
    return out.reshape(nb * BM)


SC_NC = 2
SC_NS = 16
NW = SC_NC * SC_NS
CHK = 128

@functools.cache
def _build_gather(rows):
    bpw = rows // NW
    nchk = bpw // CHK
    mesh = plsc.VectorSubcoreMesh(core_axis_name="c", subcore_axis_name="s")

    @functools.partial(
        pl.kernel, mesh=mesh,
        out_type=jax.ShapeDtypeStruct((rows, D), jnp.float32),
        scratch_types=[
            pltpu.VMEM((nchk, CHK), jnp.int32),
            pltpu.VMEM((bpw, D), jnp.float32),
            pltpu.SemaphoreType.DMA,
        ],
    )
    def _gather(idx_hbm, table_hbm, out_hbm, idx_v, rows_v, sem):
        wid = lax.axis_index("s") * SC_NC + lax.axis_index("c")
        base = wid * bpw
        pltpu.sync_copy(idx_hbm.at[pl.ds(wid * nchk, nchk)], idx_v)
        copies = [
            pltpu.async_copy(table_hbm.at[idx_v.at[j]],
                             rows_v.at[pl.ds(j * CHK, CHK)], sem)
            for j in range(nchk)
        ]
        for cp in copies:
            cp.wait()
        pltpu.sync_copy(rows_v, out_hbm.at[pl.ds(base, bpw)])

    return _gather


BC = 2048
NBC = B // BC


def _finish_core(z_ref, n_ref, v_ref, zq_ref, dist_ref):
    z = z_ref[...]
    d = n_ref[...] - z
    d2 = jnp.sum(d * d, axis=1, keepdims=True)
    dist = jnp.sqrt(d2)
    vd = v_ref[...] + d
    vn = jnp.sqrt(jnp.sum(vd * vd, axis=1, keepdims=True)) + 1e-8
    zq_ref[...] = z + vd * (dist / vn)
    dist_ref[...] = dist.reshape(BC)
    return jnp.sum(d2)


def _finish_body(z_ref, n_ref, v_ref, zq_ref, dist_ref, acc_ref):
    i = pl.program_id(0)
    s = _finish_core(z_ref, n_ref, v_ref, zq_ref, dist_ref)
    prev = jnp.where(i == 0, 0.0, acc_ref[0, 0])
    acc_ref[...] = jnp.full((1, 1), prev + s, jnp.float32)


def _finish_body_alias(z_ref, n_ref, v_ref, zqp_ref, dp_ref, acc0_ref,
                       zq_ref, dist_ref, acc_ref):
    del zqp_ref, dp_ref
    i = pl.program_id(0)
    s = _finish_core(z_ref, n_ref, v_ref, zq_ref, dist_ref)
    prev = jnp.where(i == 0, 0.0, acc_ref[0, 0])
    tot = prev + s
    tot = jnp.where(i == pl.num_programs(0) - 1,
                    (tot + acc0_ref[0, 0]) * jnp.float32(1.0 / (B * D)), tot)
    acc_ref[...] = jnp.full((1, 1), tot, jnp.float32)


def _finish_call(z, nearest, v, off, prev=None):
    half_rows = nearest.shape[0]
    zmap = lambda i, o=off: (i + o, 0)
    in_specs = [pl.BlockSpec((BC, D), zmap),
                pl.BlockSpec((BC, D), lambda i: (i, 0)),
                pl.BlockSpec((BC, D), zmap)]
    args = [z, nearest, v]
    aliases = {}
    body = _finish_body
    if prev is not None:
        in_specs += [pl.BlockSpec(memory_space=pl.ANY),
                     pl.BlockSpec(memory_space=pl.ANY),
                     pl.BlockSpec((1, 1), lambda i: (0, 0))]
        args += [prev[0], prev[1], prev[2]]
        aliases = {3: 0, 4: 1}
        body = _finish_body_alias
    return pl.pallas_call(
        body,
        grid=(half_rows // BC,),
        in_specs=in_specs,
        out_specs=[pl.BlockSpec((BC, D), zmap),
                   pl.BlockSpec((BC,), lambda i, o=off: (i + o,)),
                   pl.BlockSpec((1, 1), lambda i: (0, 0))],
        out_shape=[jax.ShapeDtypeStruct((B, D), jnp.float32),
                   jax.ShapeDtypeStruct((B,), jnp.float32),
                   jax.ShapeDtypeStruct((1, 1), jnp.float32)],
        input_output_aliases=aliases,
        compiler_params=pltpu.CompilerParams(
            dimension_semantics=("arbitrary",)),
    )(*args)


HALVES = 2
HR = B // HALVES


def kernel(z, codebook, v):
    idx_h = [_argmin_call(z, codebook, h * (HR // BM), HR // BM)
             for h in range(HALVES)]
    gather = _build_gather(HR)
    near_h = [gather(idx_h[h].reshape(HR // CHK, CHK), codebook)
              for h in range(HALVES)]
    fin0 = _finish_call(z, near_h[0], v, 0)
    z_q, dist, acc1 = _finish_call(z, near_h[1], v, HR // BC, fin0)
    indices = jnp.concatenate(idx_h, axis=0)
    commit_loss = acc1[0, 0]
    return (z_q, indices, dist, commit_loss)
